# baseline (speedup 1.0000x reference)
import functools

import jax
import jax.numpy as jnp
from jax import lax
from jax.experimental import pallas as pl
from jax.experimental.pallas import tpu as pltpu
from jax.experimental.pallas import tpu_sc as plsc

N = 10000
E = 160000
H = 4
D = 128
NT = D // 16

NC, NS = 2, 16
NW = NC * NS
NPS = 320
BLK = 128
EB = 2000
OUTW = H * D
FLROWS = 16
FLW = FLROWS * OUTW


def _celu3(x):
    return jnp.where(x > 0, x, 3.0 * jnp.exp(x * (1.0 / 3.0)) - 3.0)


def _a1_body(f_ref, w_ref, o_ref):
    x = jnp.dot(f_ref[...], w_ref[...], preferred_element_type=jnp.float32,
                precision=lax.Precision.HIGHEST)
    o_ref[...] = _celu3(x)


def _a1_tc(features, w16):
    return pl.pallas_call(
        _a1_body,
        grid=(10,),
        in_specs=[
            pl.BlockSpec((N // 10, D), lambda i: (i, 0)),
            pl.BlockSpec((D, 16), lambda i: (0, 0)),
        ],
        out_specs=pl.BlockSpec((N // 10, 16), lambda i: (i, 0)),
        out_shape=jax.ShapeDtypeStruct((N, 16), jnp.float32),
    )(features, w16)


def _pre_body(e_ref, w_ref, eft_ref, a2_ref):
    eft = _celu3(e_ref[...])
    eft_ref[...] = eft
    a2_ref[...] = jnp.dot(eft, w_ref[...], preferred_element_type=jnp.float32,
                          precision=lax.Precision.HIGHEST)


def _pre_tc(emb, w16):
    return pl.pallas_call(
        _pre_body,
        grid=(E // EB,),
        in_specs=[
            pl.BlockSpec((EB, D), lambda i: (i, 0)),
            pl.BlockSpec((D, 16), lambda i: (0, 0)),
        ],
        out_specs=[
            pl.BlockSpec((EB, D), lambda i: (i, 0)),
            pl.BlockSpec((EB, 16), lambda i: (i, 0)),
        ],
        out_shape=[
            jax.ShapeDtypeStruct((E, D), jnp.float32),
            jax.ShapeDtypeStruct((E, 16), jnp.float32),
        ],
    )(emb, w16)


_MESH = plsc.VectorSubcoreMesh(
    core_axis_name="c", subcore_axis_name="s", num_cores=NC, num_subcores=NS
)


@functools.partial(
    pl.kernel,
    out_type=jax.ShapeDtypeStruct((N * OUTW,), jnp.float32),
    mesh=_MESH,
    compiler_params=pltpu.CompilerParams(needs_layout_passes=False),
    scratch_types=[
        pltpu.VMEM((NPS * 16,), jnp.float32),
        pltpu.VMEM((2 * BLK,), jnp.int32),
        pltpu.VMEM((2 * BLK * D,), jnp.float32),
        pltpu.VMEM((2 * BLK * 16,), jnp.float32),
        pltpu.VMEM((2 * FLW,), jnp.float32),
        pltpu.VMEM((16,), jnp.int32),
        pltpu.SemaphoreType.DMA,
        pltpu.SemaphoreType.DMA,
        pltpu.SemaphoreType.DMA,
        pltpu.SemaphoreType.DMA,
        pltpu.SemaphoreType.DMA,
        pltpu.SemaphoreType.DMA,
    ],
)
def _sc_main(a1_hbm, seg_hbm, eft_hbm, a2_hbm, out_hbm,
             a1_v, seg_v, eft_v, a2_v, stage_v, probe_v,
             semE0, semE1, semS0, semS1, semF0, semF1):
    wid = lax.axis_index("c") * NS + lax.axis_index("s")
    n0 = wid * NPS
    num = jnp.minimum(NPS, N - n0)
    n1 = n0 + num

    pltpu.sync_copy(a1_hbm.at[pl.ds(pl.multiple_of(n0 * 16, 16), NPS * 16)], a1_v)

    lanes = lax.iota(jnp.int32, 16)
    idxc = [jnp.full((16,), h, jnp.int32) for h in range(H)]
    _dn = lax.GatherDimensionNumbers(
        offset_dims=(), collapsed_slice_dims=(0,), start_index_map=(0,))
    zero16 = jnp.zeros((16,), jnp.float32)
    d0 = zero16
    o0 = tuple(zero16 for _ in range(H * NT))
    m0 = jnp.full((16,), -1e30, jnp.float32)

    def bc(v, h):
        return lax.gather(v, idxc[h][:, None], dimension_numbers=_dn,
                          slice_sizes=(1,),
                          mode=lax.GatherScatterMode.PROMISE_IN_BOUNDS)

    def ext_i32(vec, lane):
        return jnp.sum(jnp.where(lanes == lane, vec, 0))

    def seg_probe(pos):
        off = jnp.minimum(pos - lax.rem(pos, 8), E - 16)
        off = pl.multiple_of(off, 8)
        pltpu.sync_copy(seg_hbm.at[pl.ds(off, 16)], probe_v)
        return ext_i32(probe_v[pl.ds(0, 16)], pos - off)

    def lower_bound(tgt):
        def b(_, st):
            lo, hi = st
            mid = (lo + hi) // 2
            v = seg_probe(mid)
            act = lo < hi
            big = v >= tgt
            lo2 = jnp.where(jnp.logical_and(act, jnp.logical_not(big)), mid + 1, lo)
            hi2 = jnp.where(jnp.logical_and(act, big), mid, hi)
            return lo2, hi2
        lo, _ = lax.fori_loop(0, 18, b, (jnp.int32(0), jnp.int32(E)))
        return lo

    r0 = lower_bound(n0)
    r1 = jnp.where(n1 >= N, jnp.int32(E), lower_bound(n1))

    def emit(n_loc, d, o):
        f = lax.shift_right_logical(n_loc, 4)
        within = jnp.bitwise_and(n_loc, 15)
        parity = jnp.bitwise_and(f, 1)

        def wait_par():
            def w0():
                pltpu.make_async_copy(
                    stage_v.at[pl.ds(0, FLW)], out_hbm.at[pl.ds(0, FLW)], semF0
                ).wait()
            def w1():
                pltpu.make_async_copy(
                    stage_v.at[pl.ds(FLW, FLW)], out_hbm.at[pl.ds(0, FLW)], semF1
                ).wait()
            lax.cond(parity == 0, w0, w1)

        lax.cond(jnp.logical_and(within == 0, f >= 2), wait_par, lambda: None)

        st = pl.multiple_of(parity * FLW + within * OUTW, OUTW)
        for h in range(H):
            dh = bc(d, h)
            inv = 1.0 / jnp.where(dh > 0, dh, 1.0)
            for t in range(NT):
                stage_v[pl.ds(st + h * D + 16 * t, 16)] = _celu3(
                    o[h * NT + t] * inv)

        def start_flush():
            dst = pl.multiple_of((n0 + n_loc - (FLROWS - 1)) * OUTW, FLW)
            def s0():
                pltpu.async_copy(
                    stage_v.at[pl.ds(0, FLW)], out_hbm.at[pl.ds(dst, FLW)], semF0)
            def s1():
                pltpu.async_copy(
                    stage_v.at[pl.ds(FLW, FLW)], out_hbm.at[pl.ds(dst, FLW)], semF1)
            lax.cond(parity == 0, s0, s1)

        lax.cond(within == FLROWS - 1, start_flush, lambda: None)

    def emit_zero(n_loc):
        emit(n_loc, d0, o0)

    BS0 = (r0 // BLK) * BLK
    nblocks = (r1 - BS0 + BLK - 1) // BLK

    def start_block(gi):
        bs = BS0 + gi * BLK
        slot = jnp.bitwise_and(gi, 1)
        def s0():
            pltpu.async_copy(eft_hbm.at[pl.ds(pl.multiple_of(bs * D, D), BLK * D)],
                             eft_v.at[pl.ds(0, BLK * D)], semE0)
            pltpu.async_copy(seg_hbm.at[pl.ds(pl.multiple_of(bs, 8), BLK)],
                             seg_v.at[pl.ds(0, BLK)], semS0)
            pltpu.async_copy(a2_hbm.at[pl.ds(pl.multiple_of(bs * 16, 16), BLK * 16)],
                             a2_v.at[pl.ds(0, BLK * 16)], semS0)
        def s1():
            pltpu.async_copy(eft_hbm.at[pl.ds(pl.multiple_of(bs * D, D), BLK * D)],
                             eft_v.at[pl.ds(BLK * D, BLK * D)], semE1)
            pltpu.async_copy(seg_hbm.at[pl.ds(pl.multiple_of(bs, 8), BLK)],
                             seg_v.at[pl.ds(BLK, BLK)], semS1)
            pltpu.async_copy(a2_hbm.at[pl.ds(pl.multiple_of(bs * 16, 16), BLK * 16)],
                             a2_v.at[pl.ds(BLK * 16, BLK * 16)], semS1)
        lax.cond(slot == 0, s0, s1)

    def wait_block(parity):
        def w0():
            pltpu.make_async_copy(eft_hbm.at[pl.ds(0, BLK * D)],
                                  eft_v.at[pl.ds(0, BLK * D)], semE0).wait()
            pltpu.make_async_copy(seg_hbm.at[pl.ds(0, BLK)],
                                  seg_v.at[pl.ds(0, BLK)], semS0).wait()
            pltpu.make_async_copy(a2_hbm.at[pl.ds(0, BLK * 16)],
                                  a2_v.at[pl.ds(0, BLK * 16)], semS0).wait()
        def w1():
            pltpu.make_async_copy(eft_hbm.at[pl.ds(0, BLK * D)],
                                  eft_v.at[pl.ds(BLK * D, BLK * D)], semE1).wait()
            pltpu.make_async_copy(seg_hbm.at[pl.ds(0, BLK)],
                                  seg_v.at[pl.ds(BLK, BLK)], semS1).wait()
            pltpu.make_async_copy(a2_hbm.at[pl.ds(0, BLK * 16)],
                                  a2_v.at[pl.ds(BLK * 16, BLK * 16)], semS1).wait()
        lax.cond(parity == 0, w0, w1)

    n_init = jnp.where(r0 < r1, seg_probe(jnp.minimum(r0, E - 1)), n0)

    lax.fori_loop(n0, n_init, lambda k, _: (emit_zero(k - n0), 0)[1], 0)

    lax.cond(nblocks > 0, lambda: start_block(jnp.int32(0)), lambda: None)

    def block_body(g, carry):
        p_in, n_in, m_in, dv_in, o_in = carry
        parity = jnp.bitwise_and(g, 1)
        wait_block(parity)
        lax.cond(g + 1 < nblocks, lambda: start_block(g + 1), lambda: None)
        bs = BS0 + g * BLK
        be = jnp.minimum(bs + BLK, r1)
        vbase = parity * (BLK * D)
        abase = parity * (BLK * 16)
        sbase = parity * BLK

        def seg_at(idx_local):
            grp = pl.multiple_of(sbase + (idx_local // 16) * 16, 16)
            return ext_i32(seg_v[pl.ds(grp, 16)], idx_local - (idx_local // 16) * 16)

        def walk_cond(st):
            return st[0] < be

        def walk_body(st):
            p, n, m, dv, o = st
            a1row = a1_v[pl.ds(pl.multiple_of((n - n0) * 16, 16), 16)]

            def scan_cond(s2):
                q, e = s2
                return jnp.logical_and(e < 0, q < be - bs)

            def scan_body(s2):
                q, _ = s2
                qa = pl.multiple_of(sbase + q, 16)
                v = seg_v[pl.ds(qa, 16)]
                inwin = jnp.logical_and(q + lanes >= p - bs, q + lanes < be - bs)
                vm = jnp.where(inwin, v, n)
                fs = jnp.max(plsc.all_reduce_ffs(vm > n))
                e2 = jnp.where(fs < 16, q + fs, -1)
                return q + 16, e2

            q0 = ((p - bs) // 16) * 16
            _, e_loc = lax.while_loop(scan_cond, scan_body, (q0, jnp.int32(-1)))
            e_abs = bs + jnp.where(e_loc >= 0, e_loc, be - bs)

            def inst_body(r, c3):
                m, dv, o = c3
                roff = pl.multiple_of(vbase + (r - bs) * D, 16)
                eft = [eft_v[pl.ds(roff + 16 * t, 16)] for t in range(NT)]
                a2row = a2_v[pl.ds(pl.multiple_of(abase + (r - bs) * 16, 16), 16)]
                av = _celu3(a1row + a2row)
                mn = jnp.maximum(m, av)
                sc = jnp.exp(m - mn)
                w = jnp.exp(av - mn)
                dv = dv * sc + w
                o = list(o)
                for h in range(H):
                    wh = bc(w, h)
                    sch = bc(sc, h)
                    for t in range(NT):
                        o[h * NT + t] = o[h * NT + t] * sch + wh * eft[t]
                return mn, dv, tuple(o)

            def pair_body(i, c3):
                m, dv, o = c3
                r = p + 2 * i
                ro1 = pl.multiple_of(vbase + (r - bs) * D, 16)
                ro2 = pl.multiple_of(vbase + (r + 1 - bs) * D, 16)
                eft1 = [eft_v[pl.ds(ro1 + 16 * t, 16)] for t in range(NT)]
                eft2 = [eft_v[pl.ds(ro2 + 16 * t, 16)] for t in range(NT)]
                a21 = a2_v[pl.ds(pl.multiple_of(abase + (r - bs) * 16, 16), 16)]
                a22 = a2_v[pl.ds(pl.multiple_of(abase + (r + 1 - bs) * 16, 16), 16)]
                av1 = _celu3(a1row + a21)
                av2 = _celu3(a1row + a22)
                mn = jnp.maximum(m, jnp.maximum(av1, av2))
                sc = jnp.exp(m - mn)
                w1 = jnp.exp(av1 - mn)
                w2 = jnp.exp(av2 - mn)
                dv = dv * sc + w1 + w2
                o = list(o)
                for h in range(H):
                    wh1 = bc(w1, h)
                    wh2 = bc(w2, h)
                    sch = bc(sc, h)
                    for t in range(NT):
                        o[h * NT + t] = (o[h * NT + t] * sch + wh1 * eft1[t]
                                         + wh2 * eft2[t])
                return mn, dv, tuple(o)

            half = lax.shift_right_logical(e_abs - p, 1)
            m, dv, o = lax.fori_loop(0, half, pair_body, (m, dv, o))
            m, dv, o = lax.fori_loop(p + 2 * half, e_abs, inst_body, (m, dv, o))

            def fin():
                emit(n - n0, dv, o)
                nn = seg_at(e_abs - bs)
                lax.fori_loop(n + 1, nn,
                              lambda k, _: (emit_zero(k - n0), 0)[1], 0)
                return nn, m0, d0, o0

            def keep():
                return n, m, dv, o

            n2, m2, dv2, o2 = lax.cond(e_abs < be, fin, keep)
            return e_abs, n2, m2, dv2, o2

        p_out, n_out, m_out, dv_out, o_out = lax.while_loop(
            walk_cond, walk_body, (jnp.maximum(p_in, bs), n_in, m_in, dv_in, o_in))
        return p_out, n_out, m_out, dv_out, o_out

    _, n_fin, _, dv_fin, o_fin = lax.fori_loop(
        0, nblocks, block_body, (r0, n_init, m0, d0, o0))

    emit(n_fin - n0, dv_fin, o_fin)
    lax.fori_loop(n_fin + 1, n1, lambda k, _: (emit_zero(k - n0), 0)[1], 0)

    pltpu.make_async_copy(
        stage_v.at[pl.ds(0, FLW)], out_hbm.at[pl.ds(0, FLW)], semF0).wait()
    pltpu.make_async_copy(
        stage_v.at[pl.ds(FLW, FLW)], out_hbm.at[pl.ds(0, FLW)], semF1).wait()


def kernel(nodes, metapath_instances, metapath_embedding, features, attn1_w, attn2):
    w16 = jnp.zeros((D, 16), jnp.float32).at[:, :H].set(attn1_w.T)
    a1 = _a1_tc(features, w16)
    aw16 = jnp.zeros((D, 16), jnp.float32).at[:, :H].set(attn2.reshape(H, D).T)
    eft, a2p = _pre_tc(metapath_embedding, aw16)
    out = _sc_main(
        a1.reshape(-1),
        metapath_instances,
        eft.reshape(-1),
        a2p.reshape(-1),
    )
    return out.reshape(N, OUTW)


# speedup vs baseline: 1.2061x; 1.2061x over previous
import functools

import jax
import jax.numpy as jnp
from jax import lax
from jax.experimental import pallas as pl
from jax.experimental.pallas import tpu as pltpu
from jax.experimental.pallas import tpu_sc as plsc

N = 10000
E = 160000
H = 4
D = 128
NT = D // 16

NC, NS = 2, 16
NW = NC * NS
NPS = 320
BLK = 128
EB = 2000
OUTW = H * D
FLROWS = 16
FLW = FLROWS * OUTW


def _celu3(x):
    return jnp.where(x > 0, x, 3.0 * jnp.exp(x * (1.0 / 3.0)) - 3.0)


def _a1_body(f_ref, w_ref, o_ref):
    x = jnp.dot(f_ref[...], w_ref[...], preferred_element_type=jnp.float32,
                precision=lax.Precision.HIGHEST)
    o_ref[...] = _celu3(x)


def _a1_tc(features, w16):
    return pl.pallas_call(
        _a1_body,
        grid=(10,),
        in_specs=[
            pl.BlockSpec((N // 10, D), lambda i: (i, 0)),
            pl.BlockSpec((D, 16), lambda i: (0, 0)),
        ],
        out_specs=pl.BlockSpec((N // 10, 16), lambda i: (i, 0)),
        out_shape=jax.ShapeDtypeStruct((N, 16), jnp.float32),
    )(features, w16)


def _pre_body(e_ref, w_ref, eft_ref, a2_ref):
    eft = _celu3(e_ref[...])
    eft_ref[...] = eft
    a2_ref[...] = jnp.dot(eft, w_ref[...], preferred_element_type=jnp.float32,
                          precision=lax.Precision.HIGHEST)


def _pre_tc(emb, w16):
    return pl.pallas_call(
        _pre_body,
        grid=(E // EB,),
        in_specs=[
            pl.BlockSpec((EB, D), lambda i: (i, 0)),
            pl.BlockSpec((D, 16), lambda i: (0, 0)),
        ],
        out_specs=[
            pl.BlockSpec((EB, D), lambda i: (i, 0)),
            pl.BlockSpec((EB, 16), lambda i: (i, 0)),
        ],
        out_shape=[
            jax.ShapeDtypeStruct((E, D), jnp.float32),
            jax.ShapeDtypeStruct((E, 16), jnp.float32),
        ],
    )(emb, w16)


_MESH = plsc.VectorSubcoreMesh(
    core_axis_name="c", subcore_axis_name="s", num_cores=NC, num_subcores=NS
)


@functools.partial(
    pl.kernel,
    out_type=jax.ShapeDtypeStruct((N * OUTW,), jnp.float32),
    mesh=_MESH,
    compiler_params=pltpu.CompilerParams(needs_layout_passes=False),
    scratch_types=[
        pltpu.VMEM((NPS * 16,), jnp.float32),
        pltpu.VMEM((2 * BLK,), jnp.int32),
        pltpu.VMEM((2 * BLK * D,), jnp.float32),
        pltpu.VMEM((2 * BLK * 16,), jnp.float32),
        pltpu.VMEM((2 * FLW,), jnp.float32),
        pltpu.VMEM((16,), jnp.int32),
        pltpu.SemaphoreType.DMA,
        pltpu.SemaphoreType.DMA,
        pltpu.SemaphoreType.DMA,
        pltpu.SemaphoreType.DMA,
        pltpu.SemaphoreType.DMA,
        pltpu.SemaphoreType.DMA,
    ],
)
def _sc_main(a1_hbm, seg_hbm, eft_hbm, a2_hbm, out_hbm,
             a1_v, seg_v, eft_v, a2_v, stage_v, probe_v,
             semE0, semE1, semS0, semS1, semF0, semF1):
    wid = lax.axis_index("c") * NS + lax.axis_index("s")
    n0 = wid * NPS
    num = jnp.minimum(NPS, N - n0)
    n1 = n0 + num

    pltpu.sync_copy(a1_hbm.at[pl.ds(pl.multiple_of(n0 * 16, 16), NPS * 16)], a1_v)

    lanes = lax.iota(jnp.int32, 16)
    idxc = [jnp.full((16,), h, jnp.int32) for h in range(H)]
    _dn = lax.GatherDimensionNumbers(
        offset_dims=(), collapsed_slice_dims=(0,), start_index_map=(0,))
    zero16 = jnp.zeros((16,), jnp.float32)
    d0 = zero16
    o0 = tuple(zero16 for _ in range(H * NT))
    m0 = jnp.full((16,), -1e30, jnp.float32)

    def bc(v, h):
        return lax.gather(v, idxc[h][:, None], dimension_numbers=_dn,
                          slice_sizes=(1,),
                          mode=lax.GatherScatterMode.PROMISE_IN_BOUNDS)

    def ext_i32(vec, lane):
        return jnp.sum(jnp.where(lanes == lane, vec, 0))

    def seg_probe(pos):
        off = jnp.minimum(pos - lax.rem(pos, 8), E - 16)
        off = pl.multiple_of(off, 8)
        pltpu.sync_copy(seg_hbm.at[pl.ds(off, 16)], probe_v)
        return ext_i32(probe_v[pl.ds(0, 16)], pos - off)

    def lower_bound(tgt):
        def b(_, st):
            lo, hi = st
            mid = (lo + hi) // 2
            v = seg_probe(mid)
            act = lo < hi
            big = v >= tgt
            lo2 = jnp.where(jnp.logical_and(act, jnp.logical_not(big)), mid + 1, lo)
            hi2 = jnp.where(jnp.logical_and(act, big), mid, hi)
            return lo2, hi2
        lo, _ = lax.fori_loop(0, 18, b, (jnp.int32(0), jnp.int32(E)))
        return lo

    r0 = lower_bound(n0)
    r1 = jnp.where(n1 >= N, jnp.int32(E), lower_bound(n1))

    def emit(n_loc, d, o):
        f = lax.shift_right_logical(n_loc, 4)
        within = jnp.bitwise_and(n_loc, 15)
        parity = jnp.bitwise_and(f, 1)

        def wait_par():
            def w0():
                pltpu.make_async_copy(
                    stage_v.at[pl.ds(0, FLW)], out_hbm.at[pl.ds(0, FLW)], semF0
                ).wait()
            def w1():
                pltpu.make_async_copy(
                    stage_v.at[pl.ds(FLW, FLW)], out_hbm.at[pl.ds(0, FLW)], semF1
                ).wait()
            lax.cond(parity == 0, w0, w1)

        lax.cond(jnp.logical_and(within == 0, f >= 2), wait_par, lambda: None)

        st = pl.multiple_of(parity * FLW + within * OUTW, OUTW)
        for h in range(H):
            dh = bc(d, h)
            inv = 1.0 / jnp.where(dh > 0, dh, 1.0)
            for t in range(NT):
                stage_v[pl.ds(st + h * D + 16 * t, 16)] = _celu3(
                    o[h * NT + t] * inv)

        def start_flush():
            dst = pl.multiple_of((n0 + n_loc - (FLROWS - 1)) * OUTW, FLW)
            def s0():
                pltpu.async_copy(
                    stage_v.at[pl.ds(0, FLW)], out_hbm.at[pl.ds(dst, FLW)], semF0)
            def s1():
                pltpu.async_copy(
                    stage_v.at[pl.ds(FLW, FLW)], out_hbm.at[pl.ds(dst, FLW)], semF1)
            lax.cond(parity == 0, s0, s1)

        lax.cond(within == FLROWS - 1, start_flush, lambda: None)

    def emit_zero(n_loc):
        emit(n_loc, d0, o0)

    BS0 = (r0 // BLK) * BLK
    nblocks = (r1 - BS0 + BLK - 1) // BLK

    def start_block(gi):
        bs = BS0 + gi * BLK
        slot = jnp.bitwise_and(gi, 1)
        def s0():
            pltpu.async_copy(eft_hbm.at[pl.ds(pl.multiple_of(bs * D, D), BLK * D)],
                             eft_v.at[pl.ds(0, BLK * D)], semE0)
            pltpu.async_copy(seg_hbm.at[pl.ds(pl.multiple_of(bs, 8), BLK)],
                             seg_v.at[pl.ds(0, BLK)], semS0)
            pltpu.async_copy(a2_hbm.at[pl.ds(pl.multiple_of(bs * 16, 16), BLK * 16)],
                             a2_v.at[pl.ds(0, BLK * 16)], semS0)
        def s1():
            pltpu.async_copy(eft_hbm.at[pl.ds(pl.multiple_of(bs * D, D), BLK * D)],
                             eft_v.at[pl.ds(BLK * D, BLK * D)], semE1)
            pltpu.async_copy(seg_hbm.at[pl.ds(pl.multiple_of(bs, 8), BLK)],
                             seg_v.at[pl.ds(BLK, BLK)], semS1)
            pltpu.async_copy(a2_hbm.at[pl.ds(pl.multiple_of(bs * 16, 16), BLK * 16)],
                             a2_v.at[pl.ds(BLK * 16, BLK * 16)], semS1)
        lax.cond(slot == 0, s0, s1)

    def wait_block(parity):
        def w0():
            pltpu.make_async_copy(eft_hbm.at[pl.ds(0, BLK * D)],
                                  eft_v.at[pl.ds(0, BLK * D)], semE0).wait()
            pltpu.make_async_copy(seg_hbm.at[pl.ds(0, BLK)],
                                  seg_v.at[pl.ds(0, BLK)], semS0).wait()
            pltpu.make_async_copy(a2_hbm.at[pl.ds(0, BLK * 16)],
                                  a2_v.at[pl.ds(0, BLK * 16)], semS0).wait()
        def w1():
            pltpu.make_async_copy(eft_hbm.at[pl.ds(0, BLK * D)],
                                  eft_v.at[pl.ds(BLK * D, BLK * D)], semE1).wait()
            pltpu.make_async_copy(seg_hbm.at[pl.ds(0, BLK)],
                                  seg_v.at[pl.ds(BLK, BLK)], semS1).wait()
            pltpu.make_async_copy(a2_hbm.at[pl.ds(0, BLK * 16)],
                                  a2_v.at[pl.ds(BLK * 16, BLK * 16)], semS1).wait()
        lax.cond(parity == 0, w0, w1)

    n_init = jnp.where(r0 < r1, seg_probe(jnp.minimum(r0, E - 1)), n0)

    lax.fori_loop(n0, n_init, lambda k, _: (emit_zero(k - n0), 0)[1], 0)

    lax.cond(nblocks > 0, lambda: start_block(jnp.int32(0)), lambda: None)

    def block_body(g, carry):
        p_in, n_in, m_in, dv_in, o_in = carry
        parity = jnp.bitwise_and(g, 1)
        wait_block(parity)
        lax.cond(g + 1 < nblocks, lambda: start_block(g + 1), lambda: None)
        bs = BS0 + g * BLK
        be = jnp.minimum(bs + BLK, r1)
        vbase = parity * (BLK * D)
        abase = parity * (BLK * 16)
        sbase = parity * BLK

        def seg_at(idx_local):
            grp = pl.multiple_of(sbase + (idx_local // 16) * 16, 16)
            return ext_i32(seg_v[pl.ds(grp, 16)], idx_local - (idx_local // 16) * 16)

        def walk_cond(st):
            return st[0] < be

        def walk_body(st):
            p, n, m, dv, o = st
            a1row = a1_v[pl.ds(pl.multiple_of((n - n0) * 16, 16), 16)]

            def scan_cond(s2):
                q, e = s2
                return jnp.logical_and(e < 0, q < be - bs)

            def scan_body(s2):
                q, _ = s2
                qa = pl.multiple_of(sbase + q, 16)
                v = seg_v[pl.ds(qa, 16)]
                inwin = jnp.logical_and(q + lanes >= p - bs, q + lanes < be - bs)
                vm = jnp.where(inwin, v, n)
                fs = jnp.max(plsc.all_reduce_ffs(vm > n))
                e2 = jnp.where(fs < 16, q + fs, -1)
                return q + 16, e2

            q0 = ((p - bs) // 16) * 16
            _, e_loc = lax.while_loop(scan_cond, scan_body, (q0, jnp.int32(-1)))
            e_abs = bs + jnp.where(e_loc >= 0, e_loc, be - bs)

            def inst_body(r, c3):
                m, dv, o = c3
                roff = pl.multiple_of(vbase + (r - bs) * D, 16)
                eft = [eft_v[pl.ds(roff + 16 * t, 16)] for t in range(NT)]
                a2row = a2_v[pl.ds(pl.multiple_of(abase + (r - bs) * 16, 16), 16)]
                av = _celu3(a1row + a2row)
                mn = jnp.maximum(m, av)
                sc = jnp.exp(m - mn)
                w = jnp.exp(av - mn)
                dv = dv * sc + w
                o = list(o)
                for h in range(H):
                    wh = bc(w, h)
                    sch = bc(sc, h)
                    for t in range(NT):
                        o[h * NT + t] = o[h * NT + t] * sch + wh * eft[t]
                return mn, dv, tuple(o)

            m, dv, o = lax.fori_loop(p, e_abs, inst_body, (m, dv, o))

            def fin():
                emit(n - n0, dv, o)
                nn = seg_at(e_abs - bs)
                lax.fori_loop(n + 1, nn,
                              lambda k, _: (emit_zero(k - n0), 0)[1], 0)
                return nn, m0, d0, o0

            def keep():
                return n, m, dv, o

            n2, m2, dv2, o2 = lax.cond(e_abs < be, fin, keep)
            return e_abs, n2, m2, dv2, o2

        p_out, n_out, m_out, dv_out, o_out = lax.while_loop(
            walk_cond, walk_body, (jnp.maximum(p_in, bs), n_in, m_in, dv_in, o_in))
        return p_out, n_out, m_out, dv_out, o_out

    _, n_fin, _, dv_fin, o_fin = lax.fori_loop(
        0, nblocks, block_body, (r0, n_init, m0, d0, o0))

    emit(n_fin - n0, dv_fin, o_fin)
    lax.fori_loop(n_fin + 1, n1, lambda k, _: (emit_zero(k - n0), 0)[1], 0)

    pltpu.make_async_copy(
        stage_v.at[pl.ds(0, FLW)], out_hbm.at[pl.ds(0, FLW)], semF0).wait()
    pltpu.make_async_copy(
        stage_v.at[pl.ds(FLW, FLW)], out_hbm.at[pl.ds(0, FLW)], semF1).wait()


def kernel(nodes, metapath_instances, metapath_embedding, features, attn1_w, attn2):
    w16 = jnp.zeros((D, 16), jnp.float32).at[:, :H].set(attn1_w.T)
    a1 = _a1_tc(features, w16)
    aw16 = jnp.zeros((D, 16), jnp.float32).at[:, :H].set(attn2.reshape(H, D).T)
    eft, a2p = _pre_tc(metapath_embedding, aw16)
    out = _sc_main(
        a1.reshape(-1),
        metapath_instances,
        eft.reshape(-1),
        a2p.reshape(-1),
    )
    return out.reshape(N, OUTW)
